# baseline (speedup 1.0000x reference)
import jax
import jax.numpy as jnp
from jax.experimental import pallas as pl
from jax.experimental.pallas import tpu as pltpu

S = 2048
D = 2048
H = 16
KVH = 4
HD = 128
BLK = 64
NB = S // BLK
GH = 128
GROUP = H // KVH
RB = 256
NRB = S // RB
SCALE = HD ** -0.5
THRESH = 1.0 / NB
NEG = -1e30


def _rot(x):
    h = x.shape[-1] // 2
    return jnp.concatenate([-x[..., h:], x[..., :h]], axis=-1)


def _proj_kernel(x_ref, w_ref, b_ref, cos_ref, sin_ref,
                 q_ref, k_ref, v_ref, qp_ref, kp_ref):
    x = x_ref[...]
    qkv = jnp.dot(x, w_ref[...], preferred_element_type=jnp.float32) + b_ref[...]
    c = cos_ref[...]
    s = sin_ref[...]
    for h in range(H):
        qh = qkv[:, h * HD:(h + 1) * HD]
        q4 = qh.reshape(RB // BLK, BLK, HD)
        qp_ref[:, h, :HD] = jnp.mean(q4, axis=1)
        qp_ref[:, h, HD:] = jnp.max(q4, axis=1)
        q_ref[h] = qh * c + _rot(qh) * s
    for g in range(KVH):
        kh = qkv[:, H * HD + g * HD: H * HD + (g + 1) * HD]
        k4 = kh.reshape(RB // BLK, BLK, HD)
        kp_ref[:, g, :HD] = jnp.mean(k4, axis=1)
        kp_ref[:, g, HD:] = jnp.max(k4, axis=1)
        k_ref[g] = kh * c + _rot(kh) * s
        v_ref[g] = qkv[:, (H + KVH) * HD + g * HD: (H + KVH) * HD + (g + 1) * HD]


def _gate_kernel(qp_ref, kp_ref, wqg_ref, wkg_ref, bc_ref, bs_ref, mask_ref):
    qg = jnp.dot(qp_ref[...].reshape(NB * H, 2 * HD), wqg_ref[...],
                 preferred_element_type=jnp.float32).reshape(NB, H, GH)
    kg = jnp.dot(kp_ref[...].reshape(NB * KVH, 2 * HD), wkg_ref[...],
                 preferred_element_type=jnp.float32).reshape(NB, KVH, GH)
    bc = bc_ref[...][:, None, :]
    bs = bs_ref[...][:, None, :]
    qg = qg * bc + _rot(qg) * bs
    kg = kg * bc + _rot(kg) * bs
    row = jax.lax.broadcasted_iota(jnp.int32, (NB, NB), 0)
    col = jax.lax.broadcasted_iota(jnp.int32, (NB, NB), 1)
    tri = row >= col
    for h in range(H):
        L = jax.lax.dot_general(qg[:, h, :], kg[:, h // GROUP, :],
                                (((1,), (1,)), ((), ())),
                                preferred_element_type=jnp.float32)
        L = L / jnp.sqrt(jnp.float32(GH))
        L = jnp.where(tri, L, NEG)
        m = jnp.max(L, axis=-1, keepdims=True)
        p = jnp.exp(L - m)
        p = p / jnp.sum(p, axis=-1, keepdims=True)
        sel = ((p >= THRESH) | (row == col)) & tri
        mask_ref[h * NB:(h + 1) * NB, 0, :] = sel.astype(jnp.int32)


def _attn_kernel(mask_ref, q_ref, k_ref, v_ref, o_ref):
    i = pl.program_id(1)
    q = q_ref[0]
    row = jax.lax.broadcasted_iota(jnp.int32, (BLK, BLK), 0) + i * BLK
    colb = jax.lax.broadcasted_iota(jnp.int32, (BLK, BLK), 1)

    def body(j, carry):
        def do(cr):
            m, l, acc = cr
            kj = k_ref[0, pl.ds(j * BLK, BLK), :]
            sc = jax.lax.dot_general(q, kj, (((1,), (1,)), ((), ())),
                                     preferred_element_type=jnp.float32) * SCALE
            sc = jnp.where(colb + j * BLK <= row, sc, NEG)
            mj = jnp.max(sc, axis=-1, keepdims=True)
            m_new = jnp.maximum(m, mj)
            alpha = jnp.exp(m - m_new)
            p = jnp.exp(sc - m_new)
            vj = v_ref[0, pl.ds(j * BLK, BLK), :]
            acc = acc * alpha + jnp.dot(p, vj, preferred_element_type=jnp.float32)
            l = l * alpha + jnp.sum(p, axis=-1, keepdims=True)
            return m_new, l, acc
        return jax.lax.cond(mask_ref[0, 0, j] > 0, do, lambda cr: cr, carry)

    m0 = jnp.full((BLK, 1), NEG, jnp.float32)
    l0 = jnp.zeros((BLK, 1), jnp.float32)
    a0 = jnp.zeros((BLK, HD), jnp.float32)
    m, l, acc = jax.lax.fori_loop(0, NB, body, (m0, l0, a0))
    o_ref[0] = acc / l


def _oproj_kernel(a_ref, w_ref, o_ref):
    acc = jnp.zeros((RB, D), jnp.float32)
    for h in range(H):
        acc += jnp.dot(a_ref[h], w_ref[h], preferred_element_type=jnp.float32)
    o_ref[...] = acc


def kernel(hidden_states, cos, sin, block_cos, block_sin, Wq, bq, Wk, bk, Wv, bv, Wo, Wqg, Wkg):
    x = hidden_states.reshape(S, D)
    Wqkv = jnp.concatenate([Wq, Wk, Wv], axis=1)
    bqkv = jnp.concatenate([bq, bk, bv])[None, :]

    q, k, v, qp, kp = pl.pallas_call(
        _proj_kernel,
        grid=(NRB,),
        in_specs=[
            pl.BlockSpec((RB, D), lambda i: (i, 0)),
            pl.BlockSpec((D, (H + 2 * KVH) * HD), lambda i: (0, 0)),
            pl.BlockSpec((1, (H + 2 * KVH) * HD), lambda i: (0, 0)),
            pl.BlockSpec((RB, HD), lambda i: (i, 0)),
            pl.BlockSpec((RB, HD), lambda i: (i, 0)),
        ],
        out_specs=[
            pl.BlockSpec((H, RB, HD), lambda i: (0, i, 0)),
            pl.BlockSpec((KVH, RB, HD), lambda i: (0, i, 0)),
            pl.BlockSpec((KVH, RB, HD), lambda i: (0, i, 0)),
            pl.BlockSpec((RB // BLK, H, 2 * HD), lambda i: (i, 0, 0)),
            pl.BlockSpec((RB // BLK, KVH, 2 * HD), lambda i: (i, 0, 0)),
        ],
        out_shape=[
            jax.ShapeDtypeStruct((H, S, HD), jnp.float32),
            jax.ShapeDtypeStruct((KVH, S, HD), jnp.float32),
            jax.ShapeDtypeStruct((KVH, S, HD), jnp.float32),
            jax.ShapeDtypeStruct((NB, H, 2 * HD), jnp.float32),
            jax.ShapeDtypeStruct((NB, KVH, 2 * HD), jnp.float32),
        ],
    )(x, Wqkv, bqkv, cos, sin)

    mask = pl.pallas_call(
        _gate_kernel,
        out_shape=jax.ShapeDtypeStruct((H * NB, 1, NB), jnp.int32),
    )(qp, kp, Wqg, Wkg, block_cos, block_sin)

    o = pl.pallas_call(
        _attn_kernel,
        grid=(H, NB),
        in_specs=[
            pl.BlockSpec((1, 1, NB), lambda h, i: (h * NB + i, 0, 0), memory_space=pltpu.SMEM),
            pl.BlockSpec((1, BLK, HD), lambda h, i: (h, i, 0)),
            pl.BlockSpec((1, S, HD), lambda h, i: (h // GROUP, 0, 0)),
            pl.BlockSpec((1, S, HD), lambda h, i: (h // GROUP, 0, 0)),
        ],
        out_specs=pl.BlockSpec((1, BLK, HD), lambda h, i: (h, i, 0)),
        out_shape=jax.ShapeDtypeStruct((H, S, HD), jnp.float32),
    )(mask, q, k, v)

    Wo3 = Wo.reshape(H, HD, D)
    out = pl.pallas_call(
        _oproj_kernel,
        grid=(NRB,),
        in_specs=[
            pl.BlockSpec((H, RB, HD), lambda i: (0, i, 0)),
            pl.BlockSpec((H, HD, D), lambda i: (0, 0, 0)),
        ],
        out_specs=pl.BlockSpec((RB, D), lambda i: (i, 0)),
        out_shape=jax.ShapeDtypeStruct((S, D), jnp.float32),
    )(o, Wo3)
    return out.reshape(1, S, D)


# speedup vs baseline: 6.9819x; 6.9819x over previous
import jax
import jax.numpy as jnp
from jax.experimental import pallas as pl
from jax.experimental.pallas import tpu as pltpu

S = 2048
D = 2048
H = 16
KVH = 4
HD = 128
BLK = 64
NB = S // BLK
GH = 128
GROUP = H // KVH
RB = 256
NRB = S // RB
TK = 256
SCALE = HD ** -0.5
THRESH = 1.0 / NB
NEG = -1e30


def _rot(x):
    h = x.shape[-1] // 2
    return jnp.concatenate([-x[..., h:], x[..., :h]], axis=-1)


def _proj_kernel(x_ref, w_ref, b_ref, cos_ref, sin_ref,
                 q_ref, k_ref, v_ref, qp_ref, kp_ref):
    x = x_ref[...]
    qkv = jnp.dot(x, w_ref[...], preferred_element_type=jnp.float32) + b_ref[...]
    c = cos_ref[...]
    s = sin_ref[...]
    for h in range(H):
        qh = qkv[:, h * HD:(h + 1) * HD]
        q4 = qh.reshape(RB // BLK, BLK, HD)
        qp_ref[:, h, :HD] = jnp.mean(q4, axis=1)
        qp_ref[:, h, HD:] = jnp.max(q4, axis=1)
        q_ref[h] = (qh * c + _rot(qh) * s).astype(jnp.bfloat16)
    for g in range(KVH):
        kh = qkv[:, H * HD + g * HD: H * HD + (g + 1) * HD]
        k4 = kh.reshape(RB // BLK, BLK, HD)
        kp_ref[:, g, :HD] = jnp.mean(k4, axis=1)
        kp_ref[:, g, HD:] = jnp.max(k4, axis=1)
        k_ref[g] = (kh * c + _rot(kh) * s).astype(jnp.bfloat16)
        v_ref[g] = qkv[:, (H + KVH) * HD + g * HD:
                       (H + KVH) * HD + (g + 1) * HD].astype(jnp.bfloat16)


def _gate_kernel(qp_ref, kp_ref, wqg_ref, wkg_ref, bc_ref, bs_ref, e_ref,
                 addm_ref):
    qg = jnp.dot(qp_ref[...].reshape(NB * H, 2 * HD), wqg_ref[...],
                 preferred_element_type=jnp.float32).reshape(NB, H, GH)
    kg = jnp.dot(kp_ref[...].reshape(NB * KVH, 2 * HD), wkg_ref[...],
                 preferred_element_type=jnp.float32).reshape(NB, KVH, GH)
    bc = bc_ref[...][:, None, :]
    bs = bs_ref[...][:, None, :]
    qg = qg * bc + _rot(qg) * bs
    kg = kg * bc + _rot(kg) * bs
    row = jax.lax.broadcasted_iota(jnp.int32, (NB, NB), 0)
    col = jax.lax.broadcasted_iota(jnp.int32, (NB, NB), 1)
    tri = row >= col
    e = e_ref[...]
    for h in range(H):
        L = jax.lax.dot_general(qg[:, h, :], kg[:, h // GROUP, :],
                                (((1,), (1,)), ((), ())),
                                preferred_element_type=jnp.float32)
        L = L / jnp.sqrt(jnp.float32(GH))
        L = jnp.where(tri, L, NEG)
        m = jnp.max(L, axis=-1, keepdims=True)
        p = jnp.exp(L - m)
        p = p / jnp.sum(p, axis=-1, keepdims=True)
        sel = ((p >= THRESH) | (row == col)) & tri
        sel_e = jnp.dot(sel.astype(jnp.float32), e,
                        preferred_element_type=jnp.float32)
        addm_ref[h * NRB:(h + 1) * NRB] = ((sel_e - 1.0) * 1e30).reshape(NRB, RB // BLK, S)


def _attn_kernel(q_ref, k_ref, v_ref, addm_ref, w_ref, o_ref):
    i4 = pl.program_id(0)
    h = pl.program_id(1)
    q = q_ref[0]
    rows = jax.lax.broadcasted_iota(jnp.int32, (RB, TK), 0) + i4 * RB
    cols = jax.lax.broadcasted_iota(jnp.int32, (RB, TK), 1)

    def tile(j, carry, diag):
        m, l, acc = carry
        kj = k_ref[0, pl.ds(j * TK, TK), :]
        sc = jax.lax.dot_general(q, kj, (((1,), (1,)), ((), ())),
                                 preferred_element_type=jnp.float32) * SCALE
        am = addm_ref[0, :, pl.ds(j * TK, TK)]
        sc = (sc.reshape(RB // BLK, BLK, TK) + am[:, None, :]).reshape(RB, TK)
        if diag:
            sc = jnp.where(cols + j * TK <= rows, sc, NEG)
        mj = jnp.max(sc, axis=-1, keepdims=True)
        m_new = jnp.maximum(m, mj)
        alpha = jnp.exp(m - m_new)
        p = jnp.exp(sc - m_new)
        l = l * alpha + jnp.sum(p, axis=-1, keepdims=True)
        vj = v_ref[0, pl.ds(j * TK, TK), :]
        acc = acc * alpha + jnp.dot(p.astype(jnp.bfloat16), vj,
                                    preferred_element_type=jnp.float32)
        return m_new, l, acc

    m0 = jnp.full((RB, 1), NEG, jnp.float32)
    l0 = jnp.zeros((RB, 1), jnp.float32)
    a0 = jnp.zeros((RB, HD), jnp.float32)
    carry = jax.lax.fori_loop(0, i4, lambda j, c: tile(j, c, False),
                              (m0, l0, a0))
    m, l, acc = tile(i4, carry, True)
    o = (acc / l).astype(jnp.bfloat16)

    @pl.when(h == 0)
    def _():
        o_ref[...] = jnp.zeros((RB, D), jnp.float32)

    o_ref[...] += jnp.dot(o, w_ref[0], preferred_element_type=jnp.float32)


def kernel(hidden_states, cos, sin, block_cos, block_sin, Wq, bq, Wk, bk, Wv, bv, Wo, Wqg, Wkg):
    x = hidden_states.reshape(S, D)
    Wqkv = jnp.concatenate([Wq, Wk, Wv], axis=1)
    bqkv = jnp.concatenate([bq, bk, bv])[None, :]

    q, k, v, qp, kp = pl.pallas_call(
        _proj_kernel,
        grid=(NRB,),
        in_specs=[
            pl.BlockSpec((RB, D), lambda i: (i, 0)),
            pl.BlockSpec((D, (H + 2 * KVH) * HD), lambda i: (0, 0)),
            pl.BlockSpec((1, (H + 2 * KVH) * HD), lambda i: (0, 0)),
            pl.BlockSpec((RB, HD), lambda i: (i, 0)),
            pl.BlockSpec((RB, HD), lambda i: (i, 0)),
        ],
        out_specs=[
            pl.BlockSpec((H, RB, HD), lambda i: (0, i, 0)),
            pl.BlockSpec((KVH, RB, HD), lambda i: (0, i, 0)),
            pl.BlockSpec((KVH, RB, HD), lambda i: (0, i, 0)),
            pl.BlockSpec((RB // BLK, H, 2 * HD), lambda i: (i, 0, 0)),
            pl.BlockSpec((RB // BLK, KVH, 2 * HD), lambda i: (i, 0, 0)),
        ],
        out_shape=[
            jax.ShapeDtypeStruct((H, S, HD), jnp.bfloat16),
            jax.ShapeDtypeStruct((KVH, S, HD), jnp.bfloat16),
            jax.ShapeDtypeStruct((KVH, S, HD), jnp.bfloat16),
            jax.ShapeDtypeStruct((NB, H, 2 * HD), jnp.float32),
            jax.ShapeDtypeStruct((NB, KVH, 2 * HD), jnp.float32),
        ],
    )(x, Wqkv, bqkv, cos, sin)

    blk_cols = (jax.lax.broadcasted_iota(jnp.int32, (NB, S), 1) // BLK ==
                jax.lax.broadcasted_iota(jnp.int32, (NB, S), 0)).astype(jnp.float32)
    addm = pl.pallas_call(
        _gate_kernel,
        out_shape=jax.ShapeDtypeStruct((H * NRB, RB // BLK, S), jnp.float32),
    )(qp, kp, Wqg, Wkg, block_cos, block_sin, blk_cols)

    Wo3 = Wo.reshape(H, HD, D).astype(jnp.bfloat16)
    out = pl.pallas_call(
        _attn_kernel,
        grid=(NRB, H),
        in_specs=[
            pl.BlockSpec((1, RB, HD), lambda i, h: (h, i, 0)),
            pl.BlockSpec((1, S, HD), lambda i, h: (h // GROUP, 0, 0)),
            pl.BlockSpec((1, S, HD), lambda i, h: (h // GROUP, 0, 0)),
            pl.BlockSpec((1, RB // BLK, S), lambda i, h: (h * NRB + i, 0, 0)),
            pl.BlockSpec((1, HD, D), lambda i, h: (h, 0, 0)),
        ],
        out_specs=pl.BlockSpec((RB, D), lambda i, h: (i, 0)),
        out_shape=jax.ShapeDtypeStruct((S, D), jnp.float32),
        compiler_params=pltpu.CompilerParams(
            dimension_semantics=("arbitrary", "arbitrary"),
        ),
    )(q, k, v, addm, Wo3)
    return out.reshape(1, S, D)


# speedup vs baseline: 7.5832x; 1.0861x over previous
import jax
import jax.numpy as jnp
from jax.experimental import pallas as pl
from jax.experimental.pallas import tpu as pltpu

S = 2048
D = 2048
H = 16
KVH = 4
HD = 128
BLK = 64
NB = S // BLK
GH = 128
GROUP = H // KVH
RB = 256
NRB = S // RB
TK = 256
SCALE = HD ** -0.5
LOG2E = 1.4426950408889634
QSCALE = SCALE * LOG2E
THRESH = 1.0 / NB
NEG = -1e30


def _rot(x):
    h = x.shape[-1] // 2
    return jnp.concatenate([-x[..., h:], x[..., :h]], axis=-1)


def _proj_kernel(x_ref, w_ref, b_ref, cos_ref, sin_ref,
                 q_ref, k_ref, v_ref, qp_ref, kp_ref):
    x = x_ref[...]
    qkv = jnp.dot(x, w_ref[...], preferred_element_type=jnp.float32) + b_ref[...]
    c = cos_ref[...]
    s = sin_ref[...]
    for h in range(H):
        qh = qkv[:, h * HD:(h + 1) * HD]
        q4 = qh.reshape(RB // BLK, BLK, HD)
        qp_ref[:, h, :HD] = jnp.mean(q4, axis=1)
        qp_ref[:, h, HD:] = jnp.max(q4, axis=1)
        q_ref[h] = ((qh * c + _rot(qh) * s) * QSCALE).astype(jnp.bfloat16)
    for g in range(KVH):
        kh = qkv[:, H * HD + g * HD: H * HD + (g + 1) * HD]
        k4 = kh.reshape(RB // BLK, BLK, HD)
        kp_ref[:, g, :HD] = jnp.mean(k4, axis=1)
        kp_ref[:, g, HD:] = jnp.max(k4, axis=1)
        k_ref[g] = (kh * c + _rot(kh) * s).astype(jnp.bfloat16)
        v_ref[g] = qkv[:, (H + KVH) * HD + g * HD:
                       (H + KVH) * HD + (g + 1) * HD].astype(jnp.bfloat16)


def _gate_kernel(qp_ref, kp_ref, wqg_ref, wkg_ref, bc_ref, bs_ref, e_ref,
                 addm_ref):
    qg = jnp.dot(qp_ref[...].reshape(NB * H, 2 * HD), wqg_ref[...],
                 preferred_element_type=jnp.float32).reshape(NB, H, GH)
    kg = jnp.dot(kp_ref[...].reshape(NB * KVH, 2 * HD), wkg_ref[...],
                 preferred_element_type=jnp.float32).reshape(NB, KVH, GH)
    bc = bc_ref[...][:, None, :]
    bs = bs_ref[...][:, None, :]
    qg = qg * bc + _rot(qg) * bs
    kg = kg * bc + _rot(kg) * bs
    row = jax.lax.broadcasted_iota(jnp.int32, (NB, NB), 0)
    col = jax.lax.broadcasted_iota(jnp.int32, (NB, NB), 1)
    tri = row >= col
    e = e_ref[...]
    for h in range(H):
        L = jax.lax.dot_general(qg[:, h, :], kg[:, h // GROUP, :],
                                (((1,), (1,)), ((), ())),
                                preferred_element_type=jnp.float32)
        L = L / jnp.sqrt(jnp.float32(GH))
        L = jnp.where(tri, L, NEG)
        m = jnp.max(L, axis=-1, keepdims=True)
        p = jnp.exp(L - m)
        p = p / jnp.sum(p, axis=-1, keepdims=True)
        sel = ((p >= THRESH) | (row == col)) & tri
        sel_e = jnp.dot(sel.astype(jnp.float32), e,
                        preferred_element_type=jnp.float32)
        addm_ref[h * NRB:(h + 1) * NRB] = ((sel_e - 1.0) * 1e30).reshape(NRB, RB // BLK, S)


def _attn_kernel(q_ref, k_ref, v_ref, addm_ref, w_ref, o_ref):
    i4 = pl.program_id(0)
    h = pl.program_id(1)
    q = q_ref[0]
    rows = jax.lax.broadcasted_iota(jnp.int32, (RB, TK), 0) + i4 * RB
    cols = jax.lax.broadcasted_iota(jnp.int32, (RB, TK), 1)

    def score(j):
        kj = k_ref[0, pl.ds(j * TK, TK), :]
        sc = jax.lax.dot_general(q, kj, (((1,), (1,)), ((), ())),
                                 preferred_element_type=jnp.float32)
        am = addm_ref[0, :, pl.ds(j * TK, TK)]
        return (sc.reshape(RB // BLK, BLK, TK) + am[:, None, :]).reshape(RB, TK)

    def process(m, l, acc, sc, j):
        mj = jnp.max(sc, axis=-1, keepdims=True)
        m_new = jnp.maximum(m, mj)
        alpha = jnp.exp2(m - m_new)
        p = jnp.exp2(sc - m_new)
        l = l * alpha + jnp.sum(p, axis=-1, keepdims=True)
        vj = v_ref[0, pl.ds(j * TK, TK), :]
        acc = acc * alpha + jnp.dot(p.astype(jnp.bfloat16), vj,
                                    preferred_element_type=jnp.float32)
        return m_new, l, acc

    m0 = jnp.full((RB, 1), NEG, jnp.float32)
    l0 = jnp.zeros((RB, 1), jnp.float32)
    a0 = jnp.zeros((RB, HD), jnp.float32)

    def body(j, carry):
        m, l, acc, sc = carry
        sc_next = score(j)
        m, l, acc = process(m, l, acc, sc, j - 1)
        return m, l, acc, sc_next

    m, l, acc, sc = jax.lax.fori_loop(1, i4 + 1, body, (m0, l0, a0, score(0)))
    sc = jnp.where(cols + i4 * TK <= rows, sc, NEG)
    m, l, acc = process(m, l, acc, sc, i4)
    o = (acc / l).astype(jnp.bfloat16)

    @pl.when(h == 0)
    def _():
        o_ref[...] = jnp.zeros((RB, D), jnp.float32)

    o_ref[...] += jnp.dot(o, w_ref[0], preferred_element_type=jnp.float32)


def kernel(hidden_states, cos, sin, block_cos, block_sin, Wq, bq, Wk, bk, Wv, bv, Wo, Wqg, Wkg):
    x = hidden_states.reshape(S, D)
    Wqkv = jnp.concatenate([Wq, Wk, Wv], axis=1)
    bqkv = jnp.concatenate([bq, bk, bv])[None, :]

    q, k, v, qp, kp = pl.pallas_call(
        _proj_kernel,
        grid=(NRB,),
        in_specs=[
            pl.BlockSpec((RB, D), lambda i: (i, 0)),
            pl.BlockSpec((D, (H + 2 * KVH) * HD), lambda i: (0, 0)),
            pl.BlockSpec((1, (H + 2 * KVH) * HD), lambda i: (0, 0)),
            pl.BlockSpec((RB, HD), lambda i: (i, 0)),
            pl.BlockSpec((RB, HD), lambda i: (i, 0)),
        ],
        out_specs=[
            pl.BlockSpec((H, RB, HD), lambda i: (0, i, 0)),
            pl.BlockSpec((KVH, RB, HD), lambda i: (0, i, 0)),
            pl.BlockSpec((KVH, RB, HD), lambda i: (0, i, 0)),
            pl.BlockSpec((RB // BLK, H, 2 * HD), lambda i: (i, 0, 0)),
            pl.BlockSpec((RB // BLK, KVH, 2 * HD), lambda i: (i, 0, 0)),
        ],
        out_shape=[
            jax.ShapeDtypeStruct((H, S, HD), jnp.bfloat16),
            jax.ShapeDtypeStruct((KVH, S, HD), jnp.bfloat16),
            jax.ShapeDtypeStruct((KVH, S, HD), jnp.bfloat16),
            jax.ShapeDtypeStruct((NB, H, 2 * HD), jnp.float32),
            jax.ShapeDtypeStruct((NB, KVH, 2 * HD), jnp.float32),
        ],
    )(x, Wqkv, bqkv, cos, sin)

    blk_cols = (jax.lax.broadcasted_iota(jnp.int32, (NB, S), 1) // BLK ==
                jax.lax.broadcasted_iota(jnp.int32, (NB, S), 0)).astype(jnp.float32)
    addm = pl.pallas_call(
        _gate_kernel,
        out_shape=jax.ShapeDtypeStruct((H * NRB, RB // BLK, S), jnp.float32),
    )(qp, kp, Wqg, Wkg, block_cos, block_sin, blk_cols)

    Wo3 = Wo.reshape(H, HD, D).astype(jnp.bfloat16)
    out = pl.pallas_call(
        _attn_kernel,
        grid=(NRB, H),
        in_specs=[
            pl.BlockSpec((1, RB, HD), lambda i, h: (h, i, 0)),
            pl.BlockSpec((1, S, HD), lambda i, h: (h // GROUP, 0, 0)),
            pl.BlockSpec((1, S, HD), lambda i, h: (h // GROUP, 0, 0)),
            pl.BlockSpec((1, RB // BLK, S), lambda i, h: (h * NRB + i, 0, 0)),
            pl.BlockSpec((1, HD, D), lambda i, h: (h, 0, 0)),
        ],
        out_specs=pl.BlockSpec((RB, D), lambda i, h: (i, 0)),
        out_shape=jax.ShapeDtypeStruct((S, D), jnp.float32),
        compiler_params=pltpu.CompilerParams(
            dimension_semantics=("arbitrary", "arbitrary"),
        ),
    )(q, k, v, addm, Wo3)
    return out.reshape(1, S, D)


# speedup vs baseline: 8.1017x; 1.0684x over previous
import jax
import jax.numpy as jnp
from jax.experimental import pallas as pl
from jax.experimental.pallas import tpu as pltpu

S = 2048
D = 2048
H = 16
KVH = 4
HD = 128
BLK = 64
NB = S // BLK
GH = 128
GROUP = H // KVH
RB = 256
NRB = S // RB
TK = 256
SCALE = HD ** -0.5
LOG2E = 1.4426950408889634
QSCALE = SCALE * LOG2E
THRESH = 1.0 / NB
NEG = -1e30


def _rot(x):
    h = x.shape[-1] // 2
    return jnp.concatenate([-x[..., h:], x[..., :h]], axis=-1)


def _proj_kernel(x_ref, w_ref, b_ref, cos_ref, sin_ref,
                 q_ref, k_ref, v_ref, qp_ref, kp_ref):
    x = x_ref[...]
    qkv = jnp.dot(x, w_ref[...], preferred_element_type=jnp.float32) + b_ref[...]
    c = cos_ref[...]
    s = sin_ref[...]
    for h in range(H):
        qh = qkv[:, h * HD:(h + 1) * HD]
        q4 = qh.reshape(RB // BLK, BLK, HD)
        qp_ref[:, h, :HD] = jnp.mean(q4, axis=1)
        qp_ref[:, h, HD:] = jnp.max(q4, axis=1)
        q_ref[h] = ((qh * c + _rot(qh) * s) * QSCALE).astype(jnp.bfloat16)
    for g in range(KVH):
        kh = qkv[:, H * HD + g * HD: H * HD + (g + 1) * HD]
        k4 = kh.reshape(RB // BLK, BLK, HD)
        kp_ref[:, g, :HD] = jnp.mean(k4, axis=1)
        kp_ref[:, g, HD:] = jnp.max(k4, axis=1)
        k_ref[g] = (kh * c + _rot(kh) * s).astype(jnp.bfloat16)
        v_ref[g] = qkv[:, (H + KVH) * HD + g * HD:
                       (H + KVH) * HD + (g + 1) * HD].astype(jnp.bfloat16)


def _gate_kernel(qp_ref, kp_ref, wqg_ref, wkg_ref, bc_ref, bs_ref, e_ref,
                 addm_ref):
    qg = jnp.dot(qp_ref[...].reshape(NB * H, 2 * HD), wqg_ref[...],
                 preferred_element_type=jnp.float32).reshape(NB, H, GH)
    kg = jnp.dot(kp_ref[...].reshape(NB * KVH, 2 * HD), wkg_ref[...],
                 preferred_element_type=jnp.float32).reshape(NB, KVH, GH)
    bc = bc_ref[...][:, None, :]
    bs = bs_ref[...][:, None, :]
    qg = qg * bc + _rot(qg) * bs
    kg = kg * bc + _rot(kg) * bs
    row = jax.lax.broadcasted_iota(jnp.int32, (NB, NB), 0)
    col = jax.lax.broadcasted_iota(jnp.int32, (NB, NB), 1)
    tri = row >= col
    e = e_ref[...]
    for h in range(H):
        L = jax.lax.dot_general(qg[:, h, :], kg[:, h // GROUP, :],
                                (((1,), (1,)), ((), ())),
                                preferred_element_type=jnp.float32)
        L = L / jnp.sqrt(jnp.float32(GH))
        L = jnp.where(tri, L, NEG)
        m = jnp.max(L, axis=-1, keepdims=True)
        p = jnp.exp(L - m)
        p = p / jnp.sum(p, axis=-1, keepdims=True)
        sel = ((p >= THRESH) | (row == col)) & tri
        key_exp = jax.lax.dot_general(e, sel.astype(jnp.float32).T,
                                      (((0,), (0,)), ((), ())),
                                      preferred_element_type=jnp.float32)
        addm_ref[h] = ((key_exp - 1.0) * 1e30).astype(jnp.bfloat16)


def _attn_kernel(q_ref, k_ref, v_ref, addm_ref, w_ref, o_ref):
    i4 = pl.program_id(0)
    h = pl.program_id(1)
    q = q_ref[0]
    e_sel = (jax.lax.broadcasted_iota(jnp.int32, (NB, RB), 0) ==
             i4 * (RB // BLK) +
             jax.lax.broadcasted_iota(jnp.int32, (NB, RB), 1) // BLK
             ).astype(jnp.bfloat16)
    keysr = jax.lax.broadcasted_iota(jnp.int32, (TK, RB), 0) + i4 * TK
    qcols = jax.lax.broadcasted_iota(jnp.int32, (TK, RB), 1) + i4 * RB

    def score(j):
        kj = k_ref[0, pl.ds(j * TK, TK), :]
        sc = jax.lax.dot_general(kj, q, (((1,), (1,)), ((), ())),
                                 preferred_element_type=jnp.float32)
        am = addm_ref[0, pl.ds(j * TK, TK), :]
        madd = jax.lax.dot_general(am, e_sel, (((1,), (0,)), ((), ())),
                                   preferred_element_type=jnp.float32)
        return sc + madd

    def process(m, l, acc, sc, j):
        mj = jnp.max(sc, axis=0, keepdims=True)
        m_new = jnp.maximum(m, mj)
        alpha = jnp.exp2(m - m_new)
        p = jnp.exp2(sc - m_new)
        l = l * alpha + jnp.sum(p, axis=0, keepdims=True)
        vj = v_ref[0, pl.ds(j * TK, TK), :]
        pv = jax.lax.dot_general(vj, p.astype(jnp.bfloat16),
                                 (((0,), (0,)), ((), ())),
                                 preferred_element_type=jnp.float32)
        acc = acc * alpha + pv
        return m_new, l, acc

    m0 = jnp.full((1, RB), NEG, jnp.float32)
    l0 = jnp.zeros((1, RB), jnp.float32)
    a0 = jnp.zeros((HD, RB), jnp.float32)

    def body(j, carry):
        m, l, acc, sc = carry
        sc_next = score(j)
        m, l, acc = process(m, l, acc, sc, j - 1)
        return m, l, acc, sc_next

    m, l, acc, sc = jax.lax.fori_loop(1, i4 + 1, body, (m0, l0, a0, score(0)))
    sc = jnp.where(keysr <= qcols, sc, NEG)
    m, l, acc = process(m, l, acc, sc, i4)
    o = (acc / l).astype(jnp.bfloat16)

    @pl.when(h == 0)
    def _():
        o_ref[...] = jnp.zeros((RB, D), jnp.float32)

    o_ref[...] += jax.lax.dot_general(o, w_ref[0], (((0,), (0,)), ((), ())),
                                      preferred_element_type=jnp.float32)


def kernel(hidden_states, cos, sin, block_cos, block_sin, Wq, bq, Wk, bk, Wv, bv, Wo, Wqg, Wkg):
    x = hidden_states.reshape(S, D)
    Wqkv = jnp.concatenate([Wq, Wk, Wv], axis=1)
    bqkv = jnp.concatenate([bq, bk, bv])[None, :]

    q, k, v, qp, kp = pl.pallas_call(
        _proj_kernel,
        grid=(NRB,),
        in_specs=[
            pl.BlockSpec((RB, D), lambda i: (i, 0)),
            pl.BlockSpec((D, (H + 2 * KVH) * HD), lambda i: (0, 0)),
            pl.BlockSpec((1, (H + 2 * KVH) * HD), lambda i: (0, 0)),
            pl.BlockSpec((RB, HD), lambda i: (i, 0)),
            pl.BlockSpec((RB, HD), lambda i: (i, 0)),
        ],
        out_specs=[
            pl.BlockSpec((H, RB, HD), lambda i: (0, i, 0)),
            pl.BlockSpec((KVH, RB, HD), lambda i: (0, i, 0)),
            pl.BlockSpec((KVH, RB, HD), lambda i: (0, i, 0)),
            pl.BlockSpec((RB // BLK, H, 2 * HD), lambda i: (i, 0, 0)),
            pl.BlockSpec((RB // BLK, KVH, 2 * HD), lambda i: (i, 0, 0)),
        ],
        out_shape=[
            jax.ShapeDtypeStruct((H, S, HD), jnp.bfloat16),
            jax.ShapeDtypeStruct((KVH, S, HD), jnp.bfloat16),
            jax.ShapeDtypeStruct((KVH, S, HD), jnp.bfloat16),
            jax.ShapeDtypeStruct((NB, H, 2 * HD), jnp.float32),
            jax.ShapeDtypeStruct((NB, KVH, 2 * HD), jnp.float32),
        ],
    )(x, Wqkv, bqkv, cos, sin)

    blk_cols = (jax.lax.broadcasted_iota(jnp.int32, (NB, S), 1) // BLK ==
                jax.lax.broadcasted_iota(jnp.int32, (NB, S), 0)).astype(jnp.float32)
    addm = pl.pallas_call(
        _gate_kernel,
        out_shape=jax.ShapeDtypeStruct((H, S, NB), jnp.bfloat16),
    )(qp, kp, Wqg, Wkg, block_cos, block_sin, blk_cols)

    Wo3 = Wo.reshape(H, HD, D).astype(jnp.bfloat16)
    out = pl.pallas_call(
        _attn_kernel,
        grid=(NRB, H),
        in_specs=[
            pl.BlockSpec((1, RB, HD), lambda i, h: (h, i, 0)),
            pl.BlockSpec((1, S, HD), lambda i, h: (h // GROUP, 0, 0)),
            pl.BlockSpec((1, S, HD), lambda i, h: (h // GROUP, 0, 0)),
            pl.BlockSpec((1, S, NB), lambda i, h: (h, 0, 0)),
            pl.BlockSpec((1, HD, D), lambda i, h: (h, 0, 0)),
        ],
        out_specs=pl.BlockSpec((RB, D), lambda i, h: (i, 0)),
        out_shape=jax.ShapeDtypeStruct((S, D), jnp.float32),
        compiler_params=pltpu.CompilerParams(
            dimension_semantics=("arbitrary", "arbitrary"),
        ),
    )(q, k, v, addm, Wo3)
    return out.reshape(1, S, D)


# speedup vs baseline: 9.3369x; 1.1525x over previous
import jax
import jax.numpy as jnp
from jax.experimental import pallas as pl
from jax.experimental.pallas import tpu as pltpu

S = 2048
D = 2048
H = 16
KVH = 4
HD = 128
BLK = 64
NB = S // BLK
GH = 128
GROUP = H // KVH
RB = 256
NRB = S // RB
TK = 256
SCALE = HD ** -0.5
LOG2E = 1.4426950408889634
QSCALE = SCALE * LOG2E
THRESH = 1.0 / NB
NEG = -1e30


def _rot(x):
    h = x.shape[-1] // 2
    return jnp.concatenate([-x[..., h:], x[..., :h]], axis=-1)


def _proj_kernel(x_ref, w_ref, b_ref, cos_ref, sin_ref,
                 q_ref, k_ref, v_ref, qp_ref, kp_ref):
    x = x_ref[...]
    qkv = jnp.dot(x, w_ref[...], preferred_element_type=jnp.float32) + b_ref[...]
    c = cos_ref[...]
    s = sin_ref[...]
    for h in range(H):
        qh = qkv[:, h * HD:(h + 1) * HD]
        q4 = qh.reshape(RB // BLK, BLK, HD)
        qp_ref[:, h, :HD] = jnp.mean(q4, axis=1)
        qp_ref[:, h, HD:] = jnp.max(q4, axis=1)
        q_ref[h] = ((qh * c + _rot(qh) * s) * QSCALE).astype(jnp.bfloat16)
    for g in range(KVH):
        kh = qkv[:, H * HD + g * HD: H * HD + (g + 1) * HD]
        k4 = kh.reshape(RB // BLK, BLK, HD)
        kp_ref[:, g, :HD] = jnp.mean(k4, axis=1)
        kp_ref[:, g, HD:] = jnp.max(k4, axis=1)
        k_ref[g] = (kh * c + _rot(kh) * s).astype(jnp.bfloat16)
        v_ref[g] = qkv[:, (H + KVH) * HD + g * HD:
                       (H + KVH) * HD + (g + 1) * HD].astype(jnp.bfloat16)


def _gate_kernel(qp_ref, kp_ref, wqg_ref, wkg_ref, bc_ref, bs_ref, e_ref,
                 addm_ref):
    qg = jnp.dot(qp_ref[...].reshape(NB * H, 2 * HD), wqg_ref[...],
                 preferred_element_type=jnp.float32).reshape(NB, H, GH)
    kg = jnp.dot(kp_ref[...].reshape(NB * KVH, 2 * HD), wkg_ref[...],
                 preferred_element_type=jnp.float32).reshape(NB, KVH, GH)
    bc = bc_ref[...][:, None, :]
    bs = bs_ref[...][:, None, :]
    qg = qg * bc + _rot(qg) * bs
    kg = kg * bc + _rot(kg) * bs
    row = jax.lax.broadcasted_iota(jnp.int32, (NB, NB), 0)
    col = jax.lax.broadcasted_iota(jnp.int32, (NB, NB), 1)
    tri = row >= col
    e = e_ref[...]
    for h in range(H):
        L = jax.lax.dot_general(qg[:, h, :], kg[:, h // GROUP, :],
                                (((1,), (1,)), ((), ())),
                                preferred_element_type=jnp.float32)
        L = L / jnp.sqrt(jnp.float32(GH))
        L = jnp.where(tri, L, NEG)
        m = jnp.max(L, axis=-1, keepdims=True)
        p = jnp.exp(L - m)
        p = p / jnp.sum(p, axis=-1, keepdims=True)
        sel = ((p >= THRESH) | (row == col)) & tri
        key_exp = jax.lax.dot_general(e, sel.astype(jnp.float32).T,
                                      (((0,), (0,)), ((), ())),
                                      preferred_element_type=jnp.float32)
        addm_ref[h] = ((key_exp - 1.0) * 1e30).astype(jnp.bfloat16)


def _attn_kernel(q_ref, k_ref, v_ref, addm_ref, o_ref):
    i4 = pl.program_id(1)
    q = q_ref[0]
    e_sel = (jax.lax.broadcasted_iota(jnp.int32, (NB, RB), 0) ==
             i4 * (RB // BLK) +
             jax.lax.broadcasted_iota(jnp.int32, (NB, RB), 1) // BLK
             ).astype(jnp.bfloat16)
    keysr = jax.lax.broadcasted_iota(jnp.int32, (TK, RB), 0) + i4 * TK
    qcols = jax.lax.broadcasted_iota(jnp.int32, (TK, RB), 1) + i4 * RB

    def score(j):
        kj = k_ref[0, pl.ds(j * TK, TK), :]
        sc = jax.lax.dot_general(kj, q, (((1,), (1,)), ((), ())),
                                 preferred_element_type=jnp.float32)
        am = addm_ref[0, pl.ds(j * TK, TK), :]
        madd = jax.lax.dot_general(am, e_sel, (((1,), (0,)), ((), ())),
                                   preferred_element_type=jnp.float32)
        return sc + madd

    def process(m, l, acc, sc, j):
        mj = jnp.max(sc, axis=0, keepdims=True)
        m_new = jnp.maximum(m, mj)
        alpha = jnp.exp2(m - m_new)
        p = jnp.exp2(sc - m_new)
        l = l * alpha + jnp.sum(p, axis=0, keepdims=True)
        vj = v_ref[0, pl.ds(j * TK, TK), :]
        pv = jax.lax.dot_general(vj, p.astype(jnp.bfloat16),
                                 (((0,), (0,)), ((), ())),
                                 preferred_element_type=jnp.float32)
        acc = acc * alpha + pv
        return m_new, l, acc

    m0 = jnp.full((1, RB), NEG, jnp.float32)
    l0 = jnp.zeros((1, RB), jnp.float32)
    a0 = jnp.zeros((HD, RB), jnp.float32)

    def body(j, carry):
        m, l, acc, sc = carry
        sc_next = score(j)
        m, l, acc = process(m, l, acc, sc, j - 1)
        return m, l, acc, sc_next

    m, l, acc, sc = jax.lax.fori_loop(1, i4 + 1, body, (m0, l0, a0, score(0)))
    sc = jnp.where(keysr <= qcols, sc, NEG)
    m, l, acc = process(m, l, acc, sc, i4)
    o_ref[0] = (acc / l).astype(jnp.bfloat16)


def _oproj_kernel(o_ref, w_ref, out_ref):
    x = o_ref[...].reshape(H * HD, RB)
    out_ref[...] = jax.lax.dot_general(x, w_ref[...], (((0,), (0,)), ((), ())),
                                       preferred_element_type=jnp.float32)


def kernel(hidden_states, cos, sin, block_cos, block_sin, Wq, bq, Wk, bk, Wv, bv, Wo, Wqg, Wkg):
    x = hidden_states.reshape(S, D)
    Wqkv = jnp.concatenate([Wq, Wk, Wv], axis=1)
    bqkv = jnp.concatenate([bq, bk, bv])[None, :]

    q, k, v, qp, kp = pl.pallas_call(
        _proj_kernel,
        grid=(NRB,),
        in_specs=[
            pl.BlockSpec((RB, D), lambda i: (i, 0)),
            pl.BlockSpec((D, (H + 2 * KVH) * HD), lambda i: (0, 0)),
            pl.BlockSpec((1, (H + 2 * KVH) * HD), lambda i: (0, 0)),
            pl.BlockSpec((RB, HD), lambda i: (i, 0)),
            pl.BlockSpec((RB, HD), lambda i: (i, 0)),
        ],
        out_specs=[
            pl.BlockSpec((H, RB, HD), lambda i: (0, i, 0)),
            pl.BlockSpec((KVH, RB, HD), lambda i: (0, i, 0)),
            pl.BlockSpec((KVH, RB, HD), lambda i: (0, i, 0)),
            pl.BlockSpec((RB // BLK, H, 2 * HD), lambda i: (i, 0, 0)),
            pl.BlockSpec((RB // BLK, KVH, 2 * HD), lambda i: (i, 0, 0)),
        ],
        out_shape=[
            jax.ShapeDtypeStruct((H, S, HD), jnp.bfloat16),
            jax.ShapeDtypeStruct((KVH, S, HD), jnp.bfloat16),
            jax.ShapeDtypeStruct((KVH, S, HD), jnp.bfloat16),
            jax.ShapeDtypeStruct((NB, H, 2 * HD), jnp.float32),
            jax.ShapeDtypeStruct((NB, KVH, 2 * HD), jnp.float32),
        ],
    )(x, Wqkv, bqkv, cos, sin)

    blk_cols = (jax.lax.broadcasted_iota(jnp.int32, (NB, S), 1) // BLK ==
                jax.lax.broadcasted_iota(jnp.int32, (NB, S), 0)).astype(jnp.float32)
    addm = pl.pallas_call(
        _gate_kernel,
        out_shape=jax.ShapeDtypeStruct((H, S, NB), jnp.bfloat16),
    )(qp, kp, Wqg, Wkg, block_cos, block_sin, blk_cols)

    oT = pl.pallas_call(
        _attn_kernel,
        grid=(H, NRB),
        in_specs=[
            pl.BlockSpec((1, RB, HD), lambda h, i: (h, i, 0)),
            pl.BlockSpec((1, S, HD), lambda h, i: (h // GROUP, 0, 0)),
            pl.BlockSpec((1, S, HD), lambda h, i: (h // GROUP, 0, 0)),
            pl.BlockSpec((1, S, NB), lambda h, i: (h, 0, 0)),
        ],
        out_specs=pl.BlockSpec((1, HD, RB), lambda h, i: (h, 0, i)),
        out_shape=jax.ShapeDtypeStruct((H, HD, S), jnp.bfloat16),
        compiler_params=pltpu.CompilerParams(
            dimension_semantics=("arbitrary", "arbitrary"),
        ),
    )(q, k, v, addm)

    Wo_bf = Wo.astype(jnp.bfloat16)
    out = pl.pallas_call(
        _oproj_kernel,
        grid=(NRB,),
        in_specs=[
            pl.BlockSpec((H, HD, RB), lambda i: (0, 0, i)),
            pl.BlockSpec((H * HD, D), lambda i: (0, 0)),
        ],
        out_specs=pl.BlockSpec((RB, D), lambda i: (i, 0)),
        out_shape=jax.ShapeDtypeStruct((S, D), jnp.float32),
    )(oT, Wo_bf)
    return out.reshape(1, S, D)
